# Initial kernel scaffold; baseline (speedup 1.0000x reference)
import jax
import jax.numpy as jnp
from jax.experimental import pallas as pl


def kernel(x, W, bias):
    raise NotImplementedError("write your pallas kernel here")


# speedup vs baseline: 3.1529x; 3.1529x over previous
import jax
import jax.numpy as jnp
from jax import lax
from jax.experimental import pallas as pl
from jax.experimental.pallas import tpu as pltpu
from jax.experimental.pallas import tpu_sc as plsc

BATCH = 16384
N_FIELDS = 26
NUM_CORES = 2
NUM_SUBCORES = 16
N_WORKERS = NUM_CORES * NUM_SUBCORES
B_PER_W = BATCH // N_WORKERS
IDX_PER_W = B_PER_W * N_FIELDS
ROW = 128
ROWS_PER_F = B_PER_W // ROW
N_ROWS = IDX_PER_W // ROW
L = 16


def _sc_body(xt_hbm, w_hbm, b_hbm, out_hbm, idx_v, vals_v, sums_v, bias_v, sem):
    wid = lax.axis_index("s") * NUM_CORES + lax.axis_index("c")
    base = wid * B_PER_W

    pltpu.sync_copy(xt_hbm.at[:, pl.ds(base, B_PER_W)], idx_v)
    pltpu.sync_copy(b_hbm, bias_v)

    w1 = w_hbm.at[0]

    def fire(j, c):
        f = j // ROWS_PER_F
        r = j % ROWS_PER_F
        pltpu.make_async_copy(
            w1.at[idx_v.at[f, pl.ds(r * ROW, ROW)]],
            vals_v.at[pl.ds(j * ROW, ROW)],
            sem,
        ).start()
        return c

    lax.fori_loop(0, N_ROWS, fire, 0)

    def drain(j, c):
        f = j // ROWS_PER_F
        r = j % ROWS_PER_F
        pltpu.make_async_copy(
            w1.at[idx_v.at[f, pl.ds(r * ROW, ROW)]],
            vals_v.at[pl.ds(j * ROW, ROW)],
            sem,
        ).wait()
        return c

    lax.fori_loop(0, N_ROWS, drain, 0)

    bias = bias_v[...]

    def reduce_chunk(c, carry):
        acc = bias
        for f in range(N_FIELDS):
            acc = acc + vals_v[pl.ds(f * B_PER_W + c * L, L)]
        e = jnp.exp(-jnp.abs(acc))
        s = jnp.where(acc >= 0, 1.0 / (1.0 + e), e / (1.0 + e))
        sums_v[pl.ds(c * L, L)] = s
        return carry

    lax.fori_loop(0, B_PER_W // L, reduce_chunk, 0)

    pltpu.sync_copy(sums_v, out_hbm.at[pl.ds(base, B_PER_W)])


def kernel(x, W, bias):
    xt = x.astype(jnp.int32).T
    w1 = W.T
    b16 = jnp.broadcast_to(bias.astype(jnp.float32), (L,))
    mesh = plsc.VectorSubcoreMesh(core_axis_name="c", subcore_axis_name="s")
    out = pl.kernel(
        _sc_body,
        out_type=jax.ShapeDtypeStruct((BATCH,), jnp.float32),
        mesh=mesh,
        compiler_params=pltpu.CompilerParams(needs_layout_passes=False),
        scratch_types=[
            pltpu.VMEM((N_FIELDS, B_PER_W), jnp.int32),
            pltpu.VMEM((IDX_PER_W,), jnp.float32),
            pltpu.VMEM((B_PER_W,), jnp.float32),
            pltpu.VMEM((L,), jnp.float32),
            pltpu.SemaphoreType.DMA,
        ],
    )(xt, w1, b16)
    return out.reshape(BATCH, 1)


# speedup vs baseline: 3.1665x; 1.0043x over previous
import jax
import jax.numpy as jnp
from jax import lax
from jax.experimental import pallas as pl
from jax.experimental.pallas import tpu as pltpu
from jax.experimental.pallas import tpu_sc as plsc

BATCH = 16384
N_FIELDS = 26
NUM_CORES = 2
NUM_SUBCORES = 16
N_WORKERS = NUM_CORES * NUM_SUBCORES
B_PER_W = BATCH // N_WORKERS
IDX_PER_W = B_PER_W * N_FIELDS
ROW = 128
ROWS_PER_F = B_PER_W // ROW
N_ROWS = IDX_PER_W // ROW
L = 16


def _sc_body(xt_hbm, w_hbm, b_hbm, out_hbm, idx_v, vals_v, sums_v, bias_v, sem):
    wid = lax.axis_index("s") * NUM_CORES + lax.axis_index("c")
    base = wid * B_PER_W

    pltpu.sync_copy(xt_hbm.at[:, pl.ds(base, B_PER_W)], idx_v)
    pltpu.sync_copy(b_hbm, bias_v)
    bias = plsc.load_gather(bias_v, [jnp.zeros((L,), jnp.int32)])

    w1 = w_hbm.at[0]

    def fire(j, c):
        f = j // ROWS_PER_F
        r = j % ROWS_PER_F
        pltpu.make_async_copy(
            w1.at[idx_v.at[f, pl.ds(r * ROW, ROW)]],
            vals_v.at[pl.ds(j * ROW, ROW)],
            sem,
        ).start()
        return c

    lax.fori_loop(0, N_ROWS, fire, 0)

    def drain(j, c):
        f = j // ROWS_PER_F
        r = j % ROWS_PER_F
        pltpu.make_async_copy(
            w1.at[idx_v.at[f, pl.ds(r * ROW, ROW)]],
            vals_v.at[pl.ds(j * ROW, ROW)],
            sem,
        ).wait()
        return c

    lax.fori_loop(0, N_ROWS, drain, 0)

    def reduce_chunk(c, carry):
        acc = bias
        for f in range(N_FIELDS):
            acc = acc + vals_v[pl.ds(f * B_PER_W + c * L, L)]
        e = jnp.exp(-jnp.abs(acc))
        s = jnp.where(acc >= 0, 1.0 / (1.0 + e), e / (1.0 + e))
        sums_v[pl.ds(c * L, L)] = s
        return carry

    lax.fori_loop(0, B_PER_W // L, reduce_chunk, 0)

    pltpu.sync_copy(sums_v, out_hbm.at[pl.ds(base, B_PER_W)])


def kernel(x, W, bias):
    xt = x.astype(jnp.int32).T
    w1 = W.T
    b1 = bias.astype(jnp.float32)
    mesh = plsc.VectorSubcoreMesh(core_axis_name="c", subcore_axis_name="s")
    out = pl.kernel(
        _sc_body,
        out_type=jax.ShapeDtypeStruct((BATCH,), jnp.float32),
        mesh=mesh,
        compiler_params=pltpu.CompilerParams(needs_layout_passes=False),
        scratch_types=[
            pltpu.VMEM((N_FIELDS, B_PER_W), jnp.int32),
            pltpu.VMEM((IDX_PER_W,), jnp.float32),
            pltpu.VMEM((B_PER_W,), jnp.float32),
            pltpu.VMEM((1,), jnp.float32),
            pltpu.SemaphoreType.DMA,
        ],
    )(xt, w1, b1)
    return out.reshape(BATCH, 1)


# speedup vs baseline: 3.2494x; 1.0262x over previous
import jax
import jax.numpy as jnp
from jax import lax
from jax.experimental import pallas as pl
from jax.experimental.pallas import tpu as pltpu
from jax.experimental.pallas import tpu_sc as plsc

BATCH = 16384
N_FIELDS = 26
NUM_CORES = 2
NUM_SUBCORES = 16
N_WORKERS = NUM_CORES * NUM_SUBCORES
B_PER_W = BATCH // N_WORKERS
IDX_PER_W = B_PER_W * N_FIELDS
ROW = 128
ROWS_PER_F = B_PER_W // ROW
N_ROWS = IDX_PER_W // ROW
L = 16


def _sc_body(xt_hbm, w_hbm, b_hbm, out_hbm, idx_v, vals_v, sums_v, bias_v, sem, isem):
    wid = lax.axis_index("s") * NUM_CORES + lax.axis_index("c")
    base = wid * B_PER_W

    def stage(f, c):
        pltpu.make_async_copy(
            xt_hbm.at[f, pl.ds(base, B_PER_W)], idx_v.at[f], isem
        ).start()
        return c

    lax.fori_loop(0, N_FIELDS, stage, 0)
    pltpu.sync_copy(b_hbm, bias_v)
    bias = plsc.load_gather(bias_v, [jnp.zeros((L,), jnp.int32)])

    w1 = w_hbm.at[0]

    def fire(f, c):
        pltpu.make_async_copy(
            xt_hbm.at[f, pl.ds(base, B_PER_W)], idx_v.at[f], isem
        ).wait()
        for r in range(ROWS_PER_F):
            pltpu.make_async_copy(
                w1.at[idx_v.at[f, pl.ds(r * ROW, ROW)]],
                vals_v.at[pl.ds(f * B_PER_W + r * ROW, ROW)],
                sem,
            ).start()
        return c

    lax.fori_loop(0, N_FIELDS, fire, 0)

    n_chunks = B_PER_W // L

    def drain(f, accs):
        for r in range(ROWS_PER_F):
            pltpu.make_async_copy(
                w1.at[idx_v.at[f, pl.ds(r * ROW, ROW)]],
                vals_v.at[pl.ds(f * B_PER_W + r * ROW, ROW)],
                sem,
            ).wait()
        return tuple(
            accs[c] + vals_v[pl.ds(f * B_PER_W + c * L, L)]
            for c in range(n_chunks)
        )

    accs = lax.fori_loop(0, N_FIELDS, drain, (bias,) * n_chunks)

    for c in range(n_chunks):
        acc = accs[c]
        e = jnp.exp(-jnp.abs(acc))
        s = jnp.where(acc >= 0, 1.0 / (1.0 + e), e / (1.0 + e))
        sums_v[pl.ds(c * L, L)] = s

    pltpu.sync_copy(sums_v, out_hbm.at[pl.ds(base, B_PER_W)])


def kernel(x, W, bias):
    xt = x.astype(jnp.int32).T
    w1 = W.T
    b1 = bias.astype(jnp.float32)
    mesh = plsc.VectorSubcoreMesh(core_axis_name="c", subcore_axis_name="s")
    out = pl.kernel(
        _sc_body,
        out_type=jax.ShapeDtypeStruct((BATCH,), jnp.float32),
        mesh=mesh,
        compiler_params=pltpu.CompilerParams(needs_layout_passes=False),
        scratch_types=[
            pltpu.VMEM((N_FIELDS, B_PER_W), jnp.int32),
            pltpu.VMEM((IDX_PER_W,), jnp.float32),
            pltpu.VMEM((B_PER_W,), jnp.float32),
            pltpu.VMEM((1,), jnp.float32),
            pltpu.SemaphoreType.DMA,
            pltpu.SemaphoreType.DMA,
        ],
    )(xt, w1, b1)
    return out.reshape(BATCH, 1)
